# Initial kernel scaffold; baseline (speedup 1.0000x reference)
import jax
import jax.numpy as jnp
from jax.experimental import pallas as pl


def kernel(x_nchw, p000, p001, p002, p003, p004, p005, p006, p007, p008, p009, p010, p011, p012, p013, p014, p015, p016, p017, p018, p019, p020, p021, p022, p023, p024, p025, p026, p027, p028, p029, p030, p031, p032, p033, p034, p035, p036, p037, p038, p039, p040, p041, p042, p043, p044, p045, p046, p047, p048, p049, p050, p051, p052, p053, p054, p055, p056, p057, p058, p059, p060, p061, p062, p063, p064, p065, p066, p067, p068, p069, p070, p071, p072, p073, p074, p075, p076, p077, p078, p079, p080, p081, p082, p083, p084, p085, p086, p087, p088, p089, p090, p091, p092, p093, p094, p095, p096, p097, p098, p099, p100, p101, p102, p103, p104, p105, p106, p107, p108, p109, p110, p111, p112, p113, p114, p115, p116, p117, p118, p119, p120, p121, p122, p123, p124, p125, p126, p127, p128, p129, p130, p131, p132, p133, p134, p135, p136, p137, p138, p139, p140, p141, p142, p143, p144, p145, p146, p147, p148, p149, p150, p151, p152, p153, p154, p155, p156, p157, p158, p159, p160, p161, p162):
    raise NotImplementedError("write your pallas kernel here")


# speedup vs baseline: 4.0644x; 4.0644x over previous
import jax
import jax.numpy as jnp
from jax.experimental import pallas as pl
from jax.experimental.pallas import tpu as pltpu

_VMEM = 32 * 1024 * 1024
_EMB = 640
_CLS = 200


def _tile_m(m, cap=512):
    t = min(m, cap)
    t -= t % 8
    while t > 8 and m % t != 0:
        t -= 8
    return max(t, 8)


def _mm_body(relu, a_ref, w_ref, s_ref, t_ref, o_ref):
    y = jnp.dot(a_ref[...], w_ref[...], preferred_element_type=jnp.float32)
    y = y * s_ref[...] + t_ref[...]
    if relu:
        y = jnp.maximum(y, 0.0)
    o_ref[...] = y.astype(o_ref.dtype)


def _mm_res_body(a_ref, w_ref, s_ref, t_ref, r_ref, o_ref):
    y = jnp.dot(a_ref[...], w_ref[...], preferred_element_type=jnp.float32)
    y = y * s_ref[...] + t_ref[...] + r_ref[...].astype(jnp.float32)
    o_ref[...] = jnp.maximum(y, 0.0).astype(o_ref.dtype)


def _matmul_affine(a, w, scale, shift, relu, residual=None):
    M, K = a.shape
    Kw, Np = w.shape
    assert Kw == K and M % 8 == 0
    tm = _tile_m(M)
    tn = min(Np, 512)
    ni, nj = M // tm, Np // tn

    in_specs = [
        pl.BlockSpec((tm, K), lambda i, j: (i, 0)),
        pl.BlockSpec((K, tn), lambda i, j: (0, j)),
        pl.BlockSpec((1, tn), lambda i, j: (0, j)),
        pl.BlockSpec((1, tn), lambda i, j: (0, j)),
    ]
    args = [a, w, scale, shift]
    if residual is not None:
        in_specs.append(pl.BlockSpec((tm, tn), lambda i, j: (i, j)))
        args.append(residual)
        body = _mm_res_body
    else:
        body = (lambda *r: _mm_body(True, *r)) if relu else (lambda *r: _mm_body(False, *r))

    return pl.pallas_call(
        body,
        out_shape=jax.ShapeDtypeStruct((M, Np), jnp.bfloat16),
        grid=(ni, nj),
        in_specs=in_specs,
        out_specs=pl.BlockSpec((tm, tn), lambda i, j: (i, j)),
        compiler_params=pltpu.CompilerParams(
            dimension_semantics=("parallel", "parallel"),
            vmem_limit_bytes=_VMEM),
    )(*args)


def _conv3x3(x, w, scale, shift, stride):
    N, H, W, C = x.shape
    Cout = w.shape[1]
    if stride == 1:
        Wp = W + 2
        xp = jnp.pad(x, ((0, 0), (1, 2), (1, 1), (0, 0)))
        R = (H + 3) * Wp
        xf = xp.reshape(N, R, C)
        Ho, Wo = H, W
        taps = [(i * Wp + j, 0) for i in range(3) for j in range(3)]
        Ct = C
    else:
        Wp = (W + 2) // 2
        Hp2 = (H + 4) // 2
        xp = jnp.pad(x, ((0, 0), (1, 3), (1, 1), (0, 0)))
        s2 = xp.reshape(N, Hp2, 2, Wp, 2, C).transpose(0, 1, 3, 2, 4, 5)
        xf = s2.reshape(N, Hp2 * Wp, 4 * C)
        R = Hp2 * Wp
        Ho, Wo = H // 2, W // 2
        taps = [((i // 2) * Wp + (j // 2), 2 * (i % 2) + (j % 2))
                for i in range(3) for j in range(3)]
        Ct = 4 * C
    M = Ho * Wp
    tn = min(Cout, 512)
    nj = Cout // tn

    def body(x_ref, w_ref, s_ref, t_ref, o_ref):
        acc = None
        for t, (off, g) in enumerate(taps):
            xs = x_ref[0, off:off + M, g * C:(g + 1) * C]
            wt = w_ref[t * C:(t + 1) * C, :]
            d = jnp.dot(xs, wt, preferred_element_type=jnp.float32)
            acc = d if acc is None else acc + d
        y = acc * s_ref[...] + t_ref[...]
        o_ref[0] = jnp.maximum(y, 0.0).astype(o_ref.dtype)

    if nj == 1:
        grid = (N,)
        x_map = lambda n: (n, 0, 0)
        w_map = lambda n: (0, 0)
        v_map = lambda n: (0, 0)
        o_map = lambda n: (n, 0, 0)
    else:
        grid = (nj, N)
        x_map = lambda j, n: (n, 0, 0)
        w_map = lambda j, n: (0, j)
        v_map = lambda j, n: (0, j)
        o_map = lambda j, n: (n, 0, j)

    out = pl.pallas_call(
        body,
        out_shape=jax.ShapeDtypeStruct((N, M, Cout), jnp.bfloat16),
        grid=grid,
        in_specs=[
            pl.BlockSpec((1, R, Ct), x_map),
            pl.BlockSpec((w.shape[0], tn), w_map),
            pl.BlockSpec((1, tn), v_map),
            pl.BlockSpec((1, tn), v_map),
        ],
        out_specs=pl.BlockSpec((1, M, tn), o_map),
        compiler_params=pltpu.CompilerParams(
            dimension_semantics=("parallel",) * len(grid),
            vmem_limit_bytes=_VMEM),
    )(xf, w, scale, shift)
    return out.reshape(N, Ho, Wp, Cout)[:, :, :Wo, :]


def _maxpool(x):
    N, H, W, C = x.shape
    Wp = (W + 2) // 2
    Hp2 = (H + 4) // 2
    xp = jnp.pad(x, ((0, 0), (1, 3), (1, 1), (0, 0)),
                 constant_values=-jnp.inf)
    s2 = xp.reshape(N, Hp2, 2, Wp, 2, C).transpose(0, 1, 3, 2, 4, 5)
    xf = s2.reshape(N, Hp2 * Wp, 4 * C)
    Ho, Wo = H // 2, W // 2
    M = Ho * Wp
    taps = [((i // 2) * Wp + (j // 2), 2 * (i % 2) + (j % 2))
            for i in range(3) for j in range(3)]

    def body(x_ref, o_ref):
        m = None
        for off, g in taps:
            xs = x_ref[0, off:off + M, g * C:(g + 1) * C]
            m = xs if m is None else jnp.maximum(m, xs)
        o_ref[0] = m

    out = pl.pallas_call(
        body,
        out_shape=jax.ShapeDtypeStruct((N, M, C), x.dtype),
        grid=(N,),
        in_specs=[pl.BlockSpec((1, Hp2 * Wp, 4 * C), lambda n: (n, 0, 0))],
        out_specs=pl.BlockSpec((1, M, C), lambda n: (n, 0, 0)),
        compiler_params=pltpu.CompilerParams(
            dimension_semantics=("parallel",), vmem_limit_bytes=_VMEM),
    )(xf)
    return out.reshape(N, Ho, Wp, C)[:, :, :Wo, :]


def _stem(x_nchw, p):
    x = jnp.transpose(x_nchw, (0, 2, 3, 1)).astype(jnp.bfloat16)
    xp = jnp.pad(x, ((0, 0), (3, 3), (3, 3), (0, 0)))
    N = x.shape[0]
    Ho = Wo = 112
    cols = [xp[:, i:i + 223:2, j:j + 223:2, :]
            for i in range(7) for j in range(7)]
    a = jnp.concatenate(cols, axis=-1).reshape(N * Ho * Wo, 147)
    out = _matmul_affine(a, p["w"][:147], p["scale"], p["shift"], relu=True)
    return out.reshape(N, Ho, Wo, out.shape[-1])


def _avg_heads(x, l1, l2):
    N, H, W, C = x.shape
    xr = x.reshape(N, H * W, C)

    def body(x_ref, w1_ref, b1_ref, w2_ref, b2_ref, emb_ref, out_ref):
        feat = jnp.mean(x_ref[...].astype(jnp.float32), axis=1)
        emb = jnp.dot(feat.astype(jnp.bfloat16), w1_ref[...],
                      preferred_element_type=jnp.float32) + b1_ref[...]
        emb_ref[...] = emb
        out_ref[...] = jnp.dot(emb.astype(jnp.bfloat16), w2_ref[...],
                               preferred_element_type=jnp.float32) + b2_ref[...]

    emb, out = pl.pallas_call(
        body,
        out_shape=(jax.ShapeDtypeStruct((N, l1["w"].shape[1]), jnp.float32),
                   jax.ShapeDtypeStruct((N, l2["w"].shape[1]), jnp.float32)),
        compiler_params=pltpu.CompilerParams(vmem_limit_bytes=_VMEM),
    )(xr, l1["w"], l1["b"], l2["w"], l2["b"])
    return emb[:, :_EMB], out[:, :_CLS]


def _unpack(ps):
    it = iter(ps)

    def conv():
        s = next(it)
        sh = next(it)
        w = next(it)
        return {"scale": s, "shift": sh, "w": w}

    layers = []
    for nb in (3, 4, 6, 3):
        blocks = []
        for bi in range(nb):
            b = {"conv1": conv(), "conv2": conv(), "conv3": conv()}
            if bi == 0:
                b["down"] = conv()
            blocks.append(b)
        layers.append(blocks)
    l1 = {"b": next(it), "w": next(it)}
    l2 = {"b": next(it), "w": next(it)}
    stem = conv()
    return layers, l1, l2, stem


def _block(x, p, stride):
    N, H, W, C = x.shape
    Ho, Wo = H // stride, W // stride
    if "down" in p:
        xd = x[:, ::stride, ::stride, :] if stride > 1 else x
        pd = p["down"]
        idf = _matmul_affine(xd.reshape(N * Ho * Wo, C), pd["w"], pd["scale"],
                             pd["shift"], relu=False)
    else:
        idf = x.reshape(N * H * W, C)
    c1 = p["conv1"]
    h = _matmul_affine(x.reshape(N * H * W, C), c1["w"], c1["scale"],
                       c1["shift"], relu=True)
    width = c1["w"].shape[1]
    h = h.reshape(N, H, W, width)
    c2 = p["conv2"]
    h = _conv3x3(h, c2["w"], c2["scale"], c2["shift"], stride)
    c3 = p["conv3"]
    out = _matmul_affine(h.reshape(N * Ho * Wo, width), c3["w"], c3["scale"],
                         c3["shift"], relu=True, residual=idf)
    return out.reshape(N, Ho, Wo, out.shape[-1])


def _forward(ps, x_nchw):
    layers, l1, l2, stem = _unpack(ps)
    x = _stem(x_nchw, stem)
    x = _maxpool(x)
    for blocks, st in zip(layers, (1, 2, 2, 2)):
        for bi, b in enumerate(blocks):
            x = _block(x, b, st if bi == 0 else 1)
    return _avg_heads(x, l1, l2)


def kernel(x_nchw, *ps):
    assert len(ps) == 163
    return _forward(ps, x_nchw)
